# baseline (speedup 1.0000x reference)
import functools

import jax
import jax.numpy as jnp
from jax import lax
from jax.experimental import pallas as pl
from jax.experimental.pallas import tpu as pltpu
from jax.experimental.pallas import tpu_sc as plsc

NUM_NODES = 10000
NPAD = 10240
FEAT = 128
BLK = 64
NUM_CORES = 2
NUM_SUBCORES = 16
NUM_TILES = NUM_CORES * NUM_SUBCORES
ROWS_PER_TILE = NPAD // NUM_SUBCORES


NBUF = 4


def _sc_partials(msg, eidx_flat):
    num_edges = msg.shape[0]
    assert num_edges % BLK == 0
    num_blocks = num_edges // BLK
    base_bpt = num_blocks // NUM_TILES
    rem = num_blocks % NUM_TILES

    mesh = plsc.VectorSubcoreMesh(core_axis_name="c", subcore_axis_name="s")

    @functools.partial(
        pl.kernel,
        mesh=mesh,
        out_type=jax.ShapeDtypeStruct((NUM_CORES, NPAD, FEAT), jnp.float32),
        scratch_types=[
            pltpu.VMEM_SHARED((NPAD, FEAT), jnp.float32),
            pltpu.VMEM((NBUF, BLK, FEAT), jnp.float32),
            pltpu.VMEM((NBUF, BLK), jnp.int32),
            pltpu.SemaphoreType.DMA((NBUF,)),
            pltpu.SemaphoreType.DMA((NBUF,)),
        ],
    )
    def sc_kernel(msg_hbm, dst_hbm, out_hbm, acc, msg_v, idx_v, msg_sem, idx_sem):
        c = lax.axis_index("c")
        s = lax.axis_index("s")
        w = c * NUM_SUBCORES + s

        @pl.loop(0, BLK)
        def _(r):
            @pl.loop(0, FEAT, step=16)
            def _(f):
                msg_v[0, r, pl.ds(f, 16)] = jnp.zeros((16,), jnp.float32)

        @pl.loop(0, ROWS_PER_TILE // BLK)
        def _(j):
            pltpu.sync_copy(
                msg_v.at[0],
                acc.at[pl.ds(s * ROWS_PER_TILE + j * BLK, BLK)],
            )

        plsc.subcore_barrier()

        base = w * base_bpt + jnp.minimum(w, rem)
        nb = base_bpt + jnp.where(w < rem, 1, 0)

        def issue(i, b):
            e0 = (base + i) * BLK
            pltpu.async_copy(dst_hbm.at[pl.ds(num_edges + e0, BLK)], idx_v.at[b], idx_sem.at[b])
            pltpu.async_copy(msg_hbm.at[pl.ds(e0, BLK)], msg_v.at[b], msg_sem.at[b])

        def wait(i, b):
            e0 = (base + i) * BLK
            pltpu.make_async_copy(dst_hbm.at[pl.ds(num_edges + e0, BLK)], idx_v.at[b], idx_sem.at[b]).wait()
            pltpu.make_async_copy(msg_hbm.at[pl.ds(e0, BLK)], msg_v.at[b], msg_sem.at[b]).wait()

        for k in range(NBUF):

            @pl.when(k < nb)
            def _(k=k):
                issue(k, k)

        def body(j, carry):
            i0 = NBUF * j
            for k in range(NBUF):
                i = i0 + k

                @pl.when(i < nb)
                def _(i=i, k=k):
                    wait(i, k)
                    pltpu.sync_copy(msg_v.at[k], acc.at[idx_v.at[k]], add=True)

                    @pl.when(i + NBUF < nb)
                    def _(i=i, k=k):
                        issue(i + NBUF, k)

            return carry

        lax.fori_loop(0, (nb + NBUF - 1) // NBUF, body, 0)

        plsc.subcore_barrier()

        pltpu.sync_copy(
            acc.at[pl.ds(s * ROWS_PER_TILE, ROWS_PER_TILE)],
            out_hbm.at[c, pl.ds(s * ROWS_PER_TILE, ROWS_PER_TILE)],
        )

    return sc_kernel(msg, eidx_flat)


def _tc_add(partials):
    rows = 1000

    def add_body(p_ref, o_ref):
        o_ref[...] = p_ref[0] + p_ref[1]

    return pl.pallas_call(
        add_body,
        grid=(NUM_NODES // rows,),
        in_specs=[pl.BlockSpec((2, rows, FEAT), lambda i: (0, i, 0))],
        out_specs=pl.BlockSpec((rows, FEAT), lambda i: (i, 0)),
        out_shape=jax.ShapeDtypeStruct((NUM_NODES, FEAT), jnp.float32),
    )(partials)


def kernel(msg, edge_index):
    flat = edge_index.reshape(-1)
    if flat.dtype != jnp.int32:
        flat = flat.astype(jnp.int32)
    partials = _sc_partials(msg, flat)
    return _tc_add(partials)


# speedup vs baseline: 1.0249x; 1.0249x over previous
import functools

import jax
import jax.numpy as jnp
from jax import lax
from jax.experimental import pallas as pl
from jax.experimental.pallas import tpu as pltpu
from jax.experimental.pallas import tpu_sc as plsc

NUM_NODES = 10000
NPAD = 10240
FEAT = 128
BLK = 64
NUM_CORES = 2
NUM_SUBCORES = 16
NUM_TILES = NUM_CORES * NUM_SUBCORES
ROWS_PER_TILE = NPAD // NUM_SUBCORES


NBUF = 4


def _sc_partials(msg, eidx_flat):
    num_edges = msg.shape[0]
    assert num_edges % BLK == 0
    num_blocks = num_edges // BLK
    base_bpt = num_blocks // NUM_TILES
    rem = num_blocks % NUM_TILES

    mesh = plsc.VectorSubcoreMesh(core_axis_name="c", subcore_axis_name="s")

    @functools.partial(
        pl.kernel,
        mesh=mesh,
        out_type=jax.ShapeDtypeStruct((NUM_CORES, NPAD, FEAT), jnp.float32),
        scratch_types=[
            pltpu.VMEM_SHARED((NPAD, FEAT), jnp.float32),
            pltpu.VMEM((NBUF, BLK, FEAT), jnp.float32),
            pltpu.VMEM((NBUF, BLK), jnp.int32),
            pltpu.SemaphoreType.DMA((NBUF,)),
            pltpu.SemaphoreType.DMA((NBUF,)),
        ],
    )
    def sc_kernel(msg_hbm, dst_hbm, out_hbm, acc, msg_v, idx_v, msg_sem, idx_sem):
        c = lax.axis_index("c")
        s = lax.axis_index("s")
        w = c * NUM_SUBCORES + s

        @pl.loop(0, BLK)
        def _(r):
            @pl.loop(0, FEAT, step=16)
            def _(f):
                msg_v[0, r, pl.ds(f, 16)] = jnp.zeros((16,), jnp.float32)

        @pl.loop(0, ROWS_PER_TILE // BLK)
        def _(j):
            pltpu.sync_copy(
                msg_v.at[0],
                acc.at[pl.ds(s * ROWS_PER_TILE + j * BLK, BLK)],
            )

        plsc.subcore_barrier()

        base = w * base_bpt + jnp.minimum(w, rem)
        nb = base_bpt + jnp.where(w < rem, 1, 0)

        def issue(i, b):
            e0 = (base + i) * BLK
            pltpu.async_copy(dst_hbm.at[pl.ds(num_edges + e0, BLK)], idx_v.at[b], idx_sem.at[b])
            pltpu.async_copy(msg_hbm.at[pl.ds(e0, BLK)], msg_v.at[b], msg_sem.at[b])

        def wait(i, b):
            e0 = (base + i) * BLK
            pltpu.make_async_copy(dst_hbm.at[pl.ds(num_edges + e0, BLK)], idx_v.at[b], idx_sem.at[b]).wait()
            pltpu.make_async_copy(msg_hbm.at[pl.ds(e0, BLK)], msg_v.at[b], msg_sem.at[b]).wait()

        for k in range(NBUF):

            @pl.when(k < nb)
            def _(k=k):
                issue(k, k)

        def body(j, carry):
            i0 = NBUF * j
            for k in range(NBUF):
                i = i0 + k

                @pl.when(i < nb)
                def _(i=i, k=k):
                    wait(i, k)
                    pltpu.sync_copy(msg_v.at[k], acc.at[idx_v.at[k]], add=True)

                    @pl.when(i + NBUF < nb)
                    def _(i=i, k=k):
                        issue(i + NBUF, k)

            return carry

        lax.fori_loop(0, (nb + NBUF - 1) // NBUF, body, 0)

        plsc.subcore_barrier()

        pltpu.sync_copy(
            acc.at[pl.ds(s * ROWS_PER_TILE, ROWS_PER_TILE)],
            out_hbm.at[c, pl.ds(s * ROWS_PER_TILE, ROWS_PER_TILE)],
        )

    return sc_kernel(msg, eidx_flat)


def _tc_add(partials):
    def add_body(p_ref, o_ref):
        o_ref[...] = p_ref[0, :NUM_NODES] + p_ref[1, :NUM_NODES]

    return pl.pallas_call(
        add_body,
        out_shape=jax.ShapeDtypeStruct((NUM_NODES, FEAT), jnp.float32),
    )(partials)


def kernel(msg, edge_index):
    flat = edge_index.reshape(-1)
    if flat.dtype != jnp.int32:
        flat = flat.astype(jnp.int32)
    partials = _sc_partials(msg, flat)
    return _tc_add(partials)


# speedup vs baseline: 1.0597x; 1.0340x over previous
import functools

import jax
import jax.numpy as jnp
from jax import lax
from jax.experimental import pallas as pl
from jax.experimental.pallas import tpu as pltpu
from jax.experimental.pallas import tpu_sc as plsc

NUM_NODES = 10000
NPAD = 10240
FEAT = 128
BLK = 64
NUM_CORES = 2
NUM_SUBCORES = 16
NUM_TILES = NUM_CORES * NUM_SUBCORES
ROWS_PER_TILE = NPAD // NUM_SUBCORES


NBUF = 4


def _sc_partials(msg, eidx_flat):
    num_edges = msg.shape[0]
    assert num_edges % BLK == 0
    num_blocks = num_edges // BLK
    base_bpt = num_blocks // NUM_TILES
    rem = num_blocks % NUM_TILES

    mesh = plsc.VectorSubcoreMesh(core_axis_name="c", subcore_axis_name="s")

    @functools.partial(
        pl.kernel,
        mesh=mesh,
        out_type=jax.ShapeDtypeStruct((NUM_CORES, NPAD, FEAT), jnp.float32),
        scratch_types=[
            pltpu.VMEM_SHARED((NPAD, FEAT), jnp.float32),
            pltpu.VMEM((NBUF, BLK, FEAT), jnp.float32),
            pltpu.VMEM((NBUF, BLK), jnp.int32),
            pltpu.VMEM((BLK, FEAT), jnp.float32),
            pltpu.SemaphoreType.DMA((NBUF,)),
            pltpu.SemaphoreType.DMA((NBUF,)),
        ],
    )
    def sc_kernel(msg_hbm, dst_hbm, out_hbm, acc, msg_v, idx_v, zbuf, msg_sem, idx_sem):
        c = lax.axis_index("c")
        s = lax.axis_index("s")
        w = c * NUM_SUBCORES + s

        base = w * base_bpt + jnp.minimum(w, rem)
        nb = base_bpt + jnp.where(w < rem, 1, 0)

        def issue(i, b):
            e0 = (base + i) * BLK
            pltpu.async_copy(dst_hbm.at[1, pl.ds(e0, BLK)], idx_v.at[b], idx_sem.at[b])
            pltpu.async_copy(msg_hbm.at[pl.ds(e0, BLK)], msg_v.at[b], msg_sem.at[b])

        def wait(i, b):
            e0 = (base + i) * BLK
            pltpu.make_async_copy(dst_hbm.at[1, pl.ds(e0, BLK)], idx_v.at[b], idx_sem.at[b]).wait()
            pltpu.make_async_copy(msg_hbm.at[pl.ds(e0, BLK)], msg_v.at[b], msg_sem.at[b]).wait()

        for k in range(NBUF):

            @pl.when(k < nb)
            def _(k=k):
                issue(k, k)

        @pl.loop(0, BLK)
        def _(r):
            @pl.loop(0, FEAT, step=16)
            def _(f):
                zbuf[r, pl.ds(f, 16)] = jnp.zeros((16,), jnp.float32)

        @pl.loop(0, ROWS_PER_TILE // BLK)
        def _(j):
            pltpu.sync_copy(
                zbuf,
                acc.at[pl.ds(s * ROWS_PER_TILE + j * BLK, BLK)],
            )

        plsc.subcore_barrier()

        def body(j, carry):
            i0 = NBUF * j
            for k in range(NBUF):
                i = i0 + k

                @pl.when(i < nb)
                def _(i=i, k=k):
                    wait(i, k)
                    pltpu.sync_copy(msg_v.at[k], acc.at[idx_v.at[k]], add=True)

                    @pl.when(i + NBUF < nb)
                    def _(i=i, k=k):
                        issue(i + NBUF, k)

            return carry

        lax.fori_loop(0, (nb + NBUF - 1) // NBUF, body, 0)

        plsc.subcore_barrier()

        pltpu.sync_copy(
            acc.at[pl.ds(s * ROWS_PER_TILE, ROWS_PER_TILE)],
            out_hbm.at[c, pl.ds(s * ROWS_PER_TILE, ROWS_PER_TILE)],
        )

    return sc_kernel(msg, eidx_flat)


def _tc_add(partials):
    def add_body(p_ref, o_ref):
        o_ref[...] = p_ref[0, :NUM_NODES] + p_ref[1, :NUM_NODES]

    return pl.pallas_call(
        add_body,
        out_shape=jax.ShapeDtypeStruct((NUM_NODES, FEAT), jnp.float32),
    )(partials)


def kernel(msg, edge_index):
    if edge_index.dtype != jnp.int32:
        edge_index = edge_index.astype(jnp.int32)
    partials = _sc_partials(msg, edge_index)
    return _tc_add(partials)
